# baseline (speedup 1.0000x reference)
import jax
import jax.numpy as jnp
from jax import lax
from jax.experimental import pallas as pl
from jax.experimental.pallas import tpu as pltpu
from jax.experimental.pallas import tpu_sc as plsc

N = 100000
N_PAD = 102400
E_BIN = 6400000
E_SYM = 1600000
STEPS = 5

NC, NS = 2, 16
NW = NC * NS
SLICE = N_PAD // NW
TSL = 1600
WIN = N_PAD // NS

CHUNK = 2048


def _even(x):
    return x + (x % 2)


CB_BIN = _even((E_BIN + NW * CHUNK - 1) // (NW * CHUNK))
CB_SYM = _even((E_SYM + NW * CHUNK - 1) // (NW * CHUNK))
TCH_BIN = NW * CB_BIN + 2
TCH_SYM = NW * CB_SYM + 2

_mesh = plsc.VectorSubcoreMesh(core_axis_name="c", subcore_axis_name="s")


def _add_slice(t0, t1):
    def add_body(j, _):
        o = pl.multiple_of(j * 16, 16)
        t0[pl.ds(o, 16)] = t0[pl.ds(o, 16)] + t1[pl.ds(o, 16)]
        return 0

    lax.fori_loop(0, TSL // 16, add_body, 0)


def _sum3_slice(b0, b1, b2, t0, t1, base):
    pltpu.sync_copy(b0.at[pl.ds(base, TSL)], t0)
    pltpu.sync_copy(b1.at[pl.ds(base, TSL)], t1)
    _add_slice(t0, t1)
    pltpu.sync_copy(b2.at[pl.ds(base, TSL)], t1)
    _add_slice(t0, t1)


def _edge_pass(e_h, n_chunks, chunk_base, w, probs_v,
               d0, lr0, d1, lr1, contrib_v, e0, e1, acc_sh):
    def fire(ci, dbuf, lrbuf, sem):
        pltpu.async_copy(e_h.at[chunk_base + ci, 0], dbuf, sem)
        pltpu.async_copy(e_h.at[chunk_base + ci, pl.ds(1, 2)], lrbuf, sem)

    def drain(dbuf, lrbuf, sem):
        pltpu.make_async_copy(e_h.at[0, 0], dbuf, sem).wait()
        pltpu.make_async_copy(e_h.at[0, pl.ds(1, 2)], lrbuf, sem).wait()

    def process(ci, dbuf, lrbuf, sem):
        drain(dbuf, lrbuf, sem)

        @plsc.parallel_loop(0, CHUNK, step=16)
        def gbody(o):
            il = lrbuf[0, pl.ds(o, 16)]
            ir = lrbuf[1, pl.ds(o, 16)]
            gl = plsc.load_gather(probs_v, [il])
            gr = plsc.load_gather(probs_v, [ir])
            contrib_v[pl.ds(o, 16)] = gl * gr * w

        pltpu.sync_copy(contrib_v, acc_sh.at[dbuf], add=True)
        fire(ci + 2, dbuf, lrbuf, sem)

    fire(0, d0, lr0, e0)
    fire(1, d1, lr1, e1)

    def pair_body(j, _):
        process(2 * j, d0, lr0, e0)
        process(2 * j + 1, d1, lr1, e1)
        return 0

    lax.fori_loop(0, n_chunks // 2, pair_body, 0)
    drain(d0, lr0, e0)
    drain(d1, lr1, e1)


def _step_body(base0, base1, base2, w16_h, eb_h, es_h,
               acc0_h, acc1_h, pr0_h, pr1_h,
               probs_v, d0, lr0, d1, lr1, contrib_v,
               t0, t1, w_v, acc_sh, e0, e1):
    cid = lax.axis_index("c")
    sid = lax.axis_index("s")
    wid = cid * NS + sid

    pltpu.sync_copy(w16_h, w_v)
    wvec = w_v[pl.ds(0, 16)]
    wb = wvec[0]
    ws = wvec[1]

    def zero_body(j, _):
        o = pl.multiple_of(j * 16, 16)
        t1[pl.ds(o, 16)] = jnp.zeros((16,), jnp.float32)
        return 0

    for h in range(WIN // TSL):
        base = sid * WIN + h * TSL
        _sum3_slice(base0, base1, base2, t0, t1, base)
        lax.fori_loop(0, TSL // 16, zero_body, 0)

        @pl.when(cid == 0)
        def _():
            pltpu.sync_copy(t0, pr0_h.at[pl.ds(base, TSL)])

        @pl.when(cid == 1)
        def _():
            pltpu.sync_copy(t0, pr1_h.at[pl.ds(base, TSL)])

        pltpu.sync_copy(t1, acc_sh.at[pl.ds(base, TSL)])

    plsc.subcore_barrier()

    @pl.when(cid == 0)
    def _():
        pltpu.sync_copy(pr0_h, probs_v)

    @pl.when(cid == 1)
    def _():
        pltpu.sync_copy(pr1_h, probs_v)

    _edge_pass(eb_h, CB_BIN, wid * CB_BIN, wb,
               probs_v, d0, lr0, d1, lr1, contrib_v, e0, e1, acc_sh)
    _edge_pass(es_h, CB_SYM, wid * CB_SYM, ws,
               probs_v, d0, lr0, d1, lr1, contrib_v, e0, e1, acc_sh)

    plsc.subcore_barrier()

    wbase = sid * WIN

    @pl.when(cid == 0)
    def _():
        pltpu.sync_copy(acc_sh.at[pl.ds(wbase, WIN)],
                        acc0_h.at[pl.ds(wbase, WIN)])

    @pl.when(cid == 1)
    def _():
        pltpu.sync_copy(acc_sh.at[pl.ds(wbase, WIN)],
                        acc1_h.at[pl.ds(wbase, WIN)])


_step = pl.kernel(
    _step_body,
    out_type=(jax.ShapeDtypeStruct((N_PAD,), jnp.float32),
              jax.ShapeDtypeStruct((N_PAD,), jnp.float32),
              jax.ShapeDtypeStruct((N_PAD,), jnp.float32),
              jax.ShapeDtypeStruct((N_PAD,), jnp.float32)),
    mesh=_mesh,
    compiler_params=pltpu.CompilerParams(needs_layout_passes=False),
    scratch_types=[
        pltpu.VMEM((N_PAD,), jnp.float32),
        pltpu.VMEM((CHUNK,), jnp.int32),
        pltpu.VMEM((2, CHUNK), jnp.int32),
        pltpu.VMEM((CHUNK,), jnp.int32),
        pltpu.VMEM((2, CHUNK), jnp.int32),
        pltpu.VMEM((CHUNK,), jnp.float32),
        pltpu.VMEM((TSL,), jnp.float32),
        pltpu.VMEM((TSL,), jnp.float32),
        pltpu.VMEM((16,), jnp.float32),
        pltpu.VMEM_SHARED((N_PAD,), jnp.float32),
        pltpu.SemaphoreType.DMA,
        pltpu.SemaphoreType.DMA,
    ],
)


def _combine_body(base0, base1, base2, out_h, t0, t1):
    cid = lax.axis_index("c")
    sid = lax.axis_index("s")
    for h in range(SLICE // TSL):
        base = (cid * NS + sid) * SLICE + h * TSL
        _sum3_slice(base0, base1, base2, t0, t1, base)
        pltpu.sync_copy(t0, out_h.at[pl.ds(base, TSL)])


_combine = pl.kernel(
    _combine_body,
    out_type=jax.ShapeDtypeStruct((N_PAD,), jnp.float32),
    mesh=_mesh,
    scratch_types=[
        pltpu.VMEM((TSL,), jnp.float32),
        pltpu.VMEM((TSL,), jnp.float32),
    ],
)


def _chunked_edges(edges, total, tch):
    padded = tch * CHUNK
    dst = jnp.concatenate(
        [edges[0], jnp.full((padded - total,), N, jnp.int32)])
    l = jnp.concatenate(
        [edges[1], jnp.zeros((padded - total,), jnp.int32)])
    r = jnp.concatenate(
        [edges[2], jnp.zeros((padded - total,), jnp.int32)])
    return jnp.stack([dst.reshape(tch, CHUNK), l.reshape(tch, CHUNK),
                      r.reshape(tch, CHUNK)], axis=1)


@jax.jit
def kernel(nullary_functions, binary_weight, symmetric_weight,
           binary_edges, symmetric_edges):
    nul = jnp.zeros((N_PAD,), jnp.float32).at[:N].set(nullary_functions)
    eb = _chunked_edges(binary_edges, E_BIN, TCH_BIN)
    es = _chunked_edges(symmetric_edges, E_SYM, TCH_SYM)
    w16 = jnp.zeros((16,), jnp.float32)
    w16 = w16.at[0].set(binary_weight).at[1].set(symmetric_weight)

    zero = jnp.zeros((N_PAD,), jnp.float32)
    acc0, acc1 = zero, zero
    for _ in range(STEPS):
        acc0, acc1, _, _ = _step(nul, acc0, acc1, w16, eb, es)
    out = _combine(nul, acc0, acc1)
    return out[:N]


# speedup vs baseline: 1.2596x; 1.2596x over previous
import functools

import jax
import jax.numpy as jnp
from jax import lax
from jax.experimental import pallas as pl
from jax.experimental.pallas import tpu as pltpu
from jax.experimental.pallas import tpu_sc as plsc

N = 100000
N_PAD = 102400
E_BIN = 6400000
E_SYM = 1600000
STEPS = 5

NC, NS = 2, 16
NW = NC * NS
SLICE = N_PAD // NW
TSL = 1600

CHUNK = 1024
def _even(x):
    return x + (x % 2)
CB_BIN = _even((E_BIN + NW * CHUNK - 1) // (NW * CHUNK))
CB_SYM = _even((E_SYM + NW * CHUNK - 1) // (NW * CHUNK))
PB = NW * CHUNK * CB_BIN + 2 * CHUNK
PS = NW * CHUNK * CB_SYM + 2 * CHUNK

_mesh = plsc.VectorSubcoreMesh(core_axis_name="c", subcore_axis_name="s")


def _add_slice(t0, t1):
    def add_body(j, _):
        o = pl.multiple_of(j * 16, 16)
        t0[pl.ds(o, 16)] = t0[pl.ds(o, 16)] + t1[pl.ds(o, 16)]
        return 0

    lax.fori_loop(0, TSL // 16, add_body, 0)


def _sum3_slice(b0, b1, b2, t0, t1, base):
    pltpu.sync_copy(b0.at[pl.ds(base, TSL)], t0)
    pltpu.sync_copy(b1.at[pl.ds(base, TSL)], t1)
    _add_slice(t0, t1)
    pltpu.sync_copy(b2.at[pl.ds(base, TSL)], t1)
    _add_slice(t0, t1)


def _edge_pass(dst_h, l_h, r_h, n_chunks, tile_base, w, probs_v,
               dst0, l0, r0, dst1, l1, r1, contrib_v, e0, e1, acc_sh):
    def fire(ci, dv, lv, rv, sem):
        off = tile_base + ci * CHUNK
        pltpu.async_copy(dst_h.at[pl.ds(off, CHUNK)], dv, sem)
        pltpu.async_copy(l_h.at[pl.ds(off, CHUNK)], lv, sem)
        pltpu.async_copy(r_h.at[pl.ds(off, CHUNK)], rv, sem)

    def drain(dv, lv, rv, sem):
        pltpu.make_async_copy(dst_h.at[pl.ds(0, CHUNK)], dv, sem).wait()
        pltpu.make_async_copy(l_h.at[pl.ds(0, CHUNK)], lv, sem).wait()
        pltpu.make_async_copy(r_h.at[pl.ds(0, CHUNK)], rv, sem).wait()

    def process(ci, dv, lv, rv, sem):
        drain(dv, lv, rv, sem)

        @plsc.parallel_loop(0, CHUNK, step=16)
        def gbody(o):
            il = lv[pl.ds(o, 16)]
            ir = rv[pl.ds(o, 16)]
            gl = plsc.load_gather(probs_v, [il])
            gr = plsc.load_gather(probs_v, [ir])
            contrib_v[pl.ds(o, 16)] = gl * gr * w

        pltpu.sync_copy(contrib_v, acc_sh.at[dv], add=True)
        fire(ci + 2, dv, lv, rv, sem)

    fire(0, dst0, l0, r0, e0)
    fire(1, dst1, l1, r1, e1)

    def pair_body(j, _):
        process(2 * j, dst0, l0, r0, e0)
        process(2 * j + 1, dst1, l1, r1, e1)
        return 0

    lax.fori_loop(0, n_chunks // 2, pair_body, 0)
    drain(dst0, l0, r0, e0)
    drain(dst1, l1, r1, e1)


def _step_body(base0, base1, base2, w16_h,
               dst_b, l_b, r_b, dst_s, l_s, r_s,
               acc0_h, acc1_h,
               probs_v, dst0, l0, r0, dst1, l1, r1, contrib_v,
               t0, t1, w_v, probs_sh, acc_sh, e0, e1):
    cid = lax.axis_index("c")
    sid = lax.axis_index("s")
    wid = cid * NS + sid

    pltpu.sync_copy(w16_h, w_v)
    wvec = w_v[pl.ds(0, 16)]
    wb = wvec[0]
    ws = wvec[1]

    def zero_body(j, _):
        o = pl.multiple_of(j * 16, 16)
        t1[pl.ds(o, 16)] = jnp.zeros((16,), jnp.float32)
        return 0

    for h in range(N_PAD // (NS * TSL)):
        base = sid * (N_PAD // NS) + h * TSL
        _sum3_slice(base0, base1, base2, t0, t1, base)
        lax.fori_loop(0, TSL // 16, zero_body, 0)
        pltpu.sync_copy(t0, probs_sh.at[pl.ds(base, TSL)])
        pltpu.sync_copy(t1, acc_sh.at[pl.ds(base, TSL)])

    plsc.subcore_barrier()

    pltpu.sync_copy(probs_sh, probs_v)

    _edge_pass(dst_b, l_b, r_b, CB_BIN, wid * CB_BIN * CHUNK, wb,
               probs_v, dst0, l0, r0, dst1, l1, r1, contrib_v, e0, e1,
               acc_sh)
    _edge_pass(dst_s, l_s, r_s, CB_SYM, wid * CB_SYM * CHUNK, ws,
               probs_v, dst0, l0, r0, dst1, l1, r1, contrib_v, e0, e1,
               acc_sh)

    plsc.subcore_barrier()

    wbase = sid * (N_PAD // NS)

    @pl.when(cid == 0)
    def _():
        pltpu.sync_copy(acc_sh.at[pl.ds(wbase, N_PAD // NS)],
                        acc0_h.at[pl.ds(wbase, N_PAD // NS)])

    @pl.when(cid == 1)
    def _():
        pltpu.sync_copy(acc_sh.at[pl.ds(wbase, N_PAD // NS)],
                        acc1_h.at[pl.ds(wbase, N_PAD // NS)])


_step = pl.kernel(
    _step_body,
    out_type=(jax.ShapeDtypeStruct((N_PAD,), jnp.float32),
              jax.ShapeDtypeStruct((N_PAD,), jnp.float32)),
    mesh=_mesh,
    compiler_params=pltpu.CompilerParams(needs_layout_passes=False),
    scratch_types=[
        pltpu.VMEM((N_PAD,), jnp.float32),
        pltpu.VMEM((CHUNK,), jnp.int32),
        pltpu.VMEM((CHUNK,), jnp.int32),
        pltpu.VMEM((CHUNK,), jnp.int32),
        pltpu.VMEM((CHUNK,), jnp.int32),
        pltpu.VMEM((CHUNK,), jnp.int32),
        pltpu.VMEM((CHUNK,), jnp.int32),
        pltpu.VMEM((CHUNK,), jnp.float32),
        pltpu.VMEM((TSL,), jnp.float32),
        pltpu.VMEM((TSL,), jnp.float32),
        pltpu.VMEM((16,), jnp.float32),
        pltpu.VMEM_SHARED((N_PAD,), jnp.float32),
        pltpu.VMEM_SHARED((N_PAD,), jnp.float32),
        pltpu.SemaphoreType.DMA,
        pltpu.SemaphoreType.DMA,
    ],
)


def _combine_body(base0, base1, base2, out_h, t0, t1):
    cid = lax.axis_index("c")
    sid = lax.axis_index("s")
    for h in range(SLICE // TSL):
        base = (cid * NS + sid) * SLICE + h * TSL
        _sum3_slice(base0, base1, base2, t0, t1, base)
        pltpu.sync_copy(t0, out_h.at[pl.ds(base, TSL)])


_combine = pl.kernel(
    _combine_body,
    out_type=jax.ShapeDtypeStruct((N_PAD,), jnp.float32),
    mesh=_mesh,
    scratch_types=[
        pltpu.VMEM((TSL,), jnp.float32),
        pltpu.VMEM((TSL,), jnp.float32),
    ],
)


def _pad_edges(edges, total, padded):
    dst = jnp.concatenate(
        [edges[0], jnp.full((padded - total,), N, jnp.int32)])
    l = jnp.concatenate(
        [edges[1], jnp.zeros((padded - total,), jnp.int32)])
    r = jnp.concatenate(
        [edges[2], jnp.zeros((padded - total,), jnp.int32)])
    return dst, l, r


@jax.jit
def kernel(nullary_functions, binary_weight, symmetric_weight,
           binary_edges, symmetric_edges):
    nul = jnp.zeros((N_PAD,), jnp.float32).at[:N].set(nullary_functions)
    dst_b, l_b, r_b = _pad_edges(binary_edges, E_BIN, PB)
    dst_s, l_s, r_s = _pad_edges(symmetric_edges, E_SYM, PS)
    w16 = jnp.zeros((16,), jnp.float32)
    w16 = w16.at[0].set(binary_weight).at[1].set(symmetric_weight)

    zero = jnp.zeros((N_PAD,), jnp.float32)
    acc0, acc1 = zero, zero
    for _ in range(STEPS):
        acc0, acc1 = _step(nul, acc0, acc1, w16,
                           dst_b, l_b, r_b, dst_s, l_s, r_s)
    out = _combine(nul, acc0, acc1)
    return out[:N]
